# baseline (speedup 1.0000x reference)
import functools

import jax
import jax.numpy as jnp
from jax import lax
from jax.experimental import pallas as pl
from jax.experimental.pallas import tpu as pltpu
from jax.experimental.pallas import tpu_sc as plsc

N_NODES = 50000
D = 256
H_EDGES = 37500
K = 8

NC = 2
NS = 16
NW = NC * NS
L = 16

E_BLK = 16
EW = 1184
H_PAD = NW * EW
NBLK = EW // E_BLK
NBLK2 = NBLK // 2
IDX_PAD = H_PAD * K
CH = IDX_PAD // NW
QD = 4
QBUF = CH + 32
CAP = 16384
SENT = 0xFFFF

NR = 128
RNG = 392
N_PAD = NR * RNG
BR = RNG + 8
QSPAN = N_PAD // QD
PIECE = 2048
SBUF = PIECE + 32 + L
G = 32

BM = 400


def _eq_body(x_hbm, idx_hbm, esum_hbm, qlist_hbm, qoff_hbm,
             idxc_v, rows0_v, rows1_v, es_v, qbuf_v, qvec_v, sg0, sg1):
    cid = lax.axis_index("c")
    sid = lax.axis_index("s")
    wid = sid * NC + cid
    base = wid * CH
    lane = lax.iota(jnp.int32, L)

    pltpu.sync_copy(idx_hbm.at[pl.ds(base, CH)], idxc_v)

    def sinit(i, c):
        qbuf_v[pl.ds(i * L, L)] = jnp.full((L,), SENT, jnp.int32)
        return c
    lax.fori_loop(0, QBUF // L + 1, sinit, 0)
    pltpu.sync_copy(qbuf_v.at[pl.ds(0, 4096)],
                    qlist_hbm.at[pl.ds(wid * CAP + QBUF, 4096)])
    pltpu.sync_copy(qbuf_v.at[pl.ds(0, CAP - QBUF - 4096)],
                    qlist_hbm.at[pl.ds(wid * CAP + QBUF + 4096,
                                       CAP - QBUF - 4096)])

    offs = [jnp.int32(0)]
    ptr = jnp.int32(0)
    for q in range(QD):
        qlo = q * QSPAN
        qhi = (q + 1) * QSPAN

        def scan(i, p):
            v = idxc_v[pl.ds(i * L, L)]
            h = (base + i * L + lane) >> 3
            m = (v >= qlo) & (v < qhi) & (h < H_EDGES)
            word = (h << 16) | v
            plsc.store_compressed(qbuf_v.at[pl.ds(p, L)], word, mask=m)
            return p + plsc.all_reduce_population_count(m)[0]

        ptr = lax.fori_loop(0, CH // L, scan, ptr)
        ptr = (ptr + 7) & ~7
        offs.append(ptr)

    qvec = jnp.zeros((L,), jnp.int32)
    for i, o in enumerate(offs):
        qvec = jnp.where(lane == i, o, qvec)
    qvec_v[pl.ds(0, L)] = qvec
    pltpu.sync_copy(qvec_v, qoff_hbm.at[pl.ds(wid * L, L)])
    pltpu.sync_copy(qbuf_v.at[pl.ds(0, QBUF)],
                    qlist_hbm.at[pl.ds(wid * CAP, QBUF)])

    rows_v = (rows0_v, rows1_v)
    sg = (sg0, sg1)

    def fire_gather(b, p):
        pltpu.async_copy(
            x_hbm.at[idxc_v.at[pl.ds(b * E_BLK * K, E_BLK * K)]],
            rows_v[p], sg[p])

    def drain_gather(p):
        pltpu.make_async_copy(
            x_hbm.at[idxc_v.at[pl.ds(0, E_BLK * K)]],
            rows_v[p], sg[p]).wait()

    def process(b, p, b2):
        if p == 0:
            fire_gather(b + 1, 1)
        else:
            @pl.when(b2 <= NBLK2 - 2)
            def _():
                fire_gather(b + 1, 0)
        drain_gather(p)

        def edge(e, c2):
            r0 = e * K
            for j in range(D // L):
                acc = rows_v[p][r0, pl.ds(j * L, L)]
                for k in range(1, K):
                    acc = acc + rows_v[p][r0 + k, pl.ds(j * L, L)]
                es_v[e, pl.ds(j * L, L)] = acc
            return c2
        lax.fori_loop(0, E_BLK, edge, 0)
        pltpu.sync_copy(es_v, esum_hbm.at[pl.ds(wid * EW + b * E_BLK, E_BLK)])

    def pair(b2, carry):
        process(2 * b2, 0, b2)
        process(2 * b2 + 1, 1, b2)
        return carry

    fire_gather(0, 0)
    lax.fori_loop(0, NBLK2, pair, 0)


_eq_call = functools.partial(
    pl.kernel,
    out_type=(
        jax.ShapeDtypeStruct((H_PAD, D), jnp.float32),
        jax.ShapeDtypeStruct((NW * CAP,), jnp.int32),
        jax.ShapeDtypeStruct((NW * L,), jnp.int32),
    ),
    mesh=plsc.VectorSubcoreMesh(core_axis_name="c", subcore_axis_name="s"),
    scratch_types=[
        pltpu.VMEM((CH,), jnp.int32),
        pltpu.VMEM((E_BLK * K, D), jnp.float32),
        pltpu.VMEM((E_BLK * K, D), jnp.float32),
        pltpu.VMEM((E_BLK, D), jnp.float32),
        pltpu.VMEM((QBUF + L,), jnp.int32),
        pltpu.VMEM((L,), jnp.int32),
        pltpu.SemaphoreType.DMA,
        pltpu.SemaphoreType.DMA,
    ],
    compiler_params=pltpu.CompilerParams(needs_layout_passes=False),
)(_eq_body)


def _a_body(qlist_hbm, qoff_hbm, esum_hbm, agg_hbm,
            qot_v, piece_v, stage_v, rows_v, gidx_v, bin_v, sgm):
    cid = lax.axis_index("c")
    sid = lax.axis_index("s")
    wid = sid * NC + cid
    lane = lax.iota(jnp.int32, L)

    pltpu.sync_copy(qoff_hbm, qot_v)

    for rr in range(QD):
        lo = (wid + rr * NW) * RNG

        def zero(i, c):
            for j in range(D // L):
                bin_v[pl.ds(i * D + j * L, L)] = jnp.zeros((L,), jnp.float32)
            return c
        lax.fori_loop(0, RNG, zero, 0)

        def src(w2, c):
            ov = qot_v[pl.ds(w2 * L, L)]
            oq = ov[rr]
            len8 = ov[rr + 1] - oq
            npiece = (len8 + PIECE - 1) // PIECE

            def piece(pi, c2):
                start = pl.multiple_of(w2 * CAP + oq + pi * PIECE, 8)
                pltpu.sync_copy(qlist_hbm.at[pl.ds(start, PIECE)], piece_v)

                def scan(i, p):
                    w = piece_v[pl.ds(i * L, L)]
                    v = w & 0xFFFF
                    m = (v >= lo) & (v < lo + RNG)
                    plsc.store_compressed(stage_v.at[pl.ds(p, L)], w, mask=m)
                    return p + plsc.all_reduce_population_count(m)[0]

                mtot = lax.fori_loop(0, PIECE // L, scan, 0)

                pad = ((lane & 7) << 16) | (lo + RNG + (lane & 7))
                full = lane < L
                for t in range(G // L):
                    plsc.store_compressed(
                        stage_v.at[pl.ds(mtot + t * L, L)], pad, mask=full)
                nb = (mtot + G - 1) // G

                def blk(bi, c3):
                    for t in range(G // L):
                        w = stage_v[pl.ds(bi * G + t * L, L)]
                        gidx_v[pl.ds(t * L, L)] = (w >> 16) & 0xFFFF
                    pltpu.async_copy(esum_hbm.at[gidx_v], rows_v, sgm).wait()

                    def row(j, c4):
                        w = stage_v[pl.ds(bi * G + (j & ~15), L)]
                        offv = (w & 0xFFFF) - lo
                        s = offv.at[jnp.full((L,), j & 15, jnp.int32)].get(
                            mode="promise_in_bounds")[0]
                        for jj in range(D // L):
                            plsc.addupdate(
                                bin_v.at[pl.ds(s * D + jj * L, L)],
                                rows_v[j, pl.ds(jj * L, L)])
                        return c4
                    lax.fori_loop(0, G, row, 0)
                    return c3

                lax.fori_loop(0, nb, blk, 0)
                return c2

            lax.fori_loop(0, npiece, piece, 0)
            return c

        lax.fori_loop(0, NW, src, 0)

        pltpu.sync_copy(bin_v.at[pl.ds(0, RNG * D)],
                        agg_hbm.at[pl.ds(lo * D, RNG * D)])


_a_call = functools.partial(
    pl.kernel,
    out_type=jax.ShapeDtypeStruct((N_PAD * D,), jnp.float32),
    mesh=plsc.VectorSubcoreMesh(core_axis_name="c", subcore_axis_name="s"),
    scratch_types=[
        pltpu.VMEM((NW * L,), jnp.int32),
        pltpu.VMEM((PIECE,), jnp.int32),
        pltpu.VMEM((SBUF,), jnp.int32),
        pltpu.VMEM((G, D), jnp.float32),
        pltpu.VMEM((G,), jnp.int32),
        pltpu.VMEM((BR * D,), jnp.float32),
        pltpu.SemaphoreType.DMA,
    ],
    compiler_params=pltpu.CompilerParams(needs_layout_passes=False),
)(_a_body)


def _matmul_body(a_ref, w_ref, o_ref):
    o_ref[...] = jnp.maximum(
        jnp.dot(a_ref[...], w_ref[...], preferred_element_type=jnp.float32),
        0.0)


_matmul_call = pl.pallas_call(
    _matmul_body,
    grid=(N_NODES // BM,),
    in_specs=[
        pl.BlockSpec((BM, D), lambda i: (i, 0)),
        pl.BlockSpec((D, D), lambda i: (0, 0)),
    ],
    out_specs=pl.BlockSpec((BM, D), lambda i: (i, 0)),
    out_shape=jax.ShapeDtypeStruct((N_NODES, D), jnp.float32),
)


def kernel(x, hyperedges, weight):
    idx_flat = jnp.pad(hyperedges.reshape(-1), (0, IDX_PAD - H_EDGES * K))
    esum, qlist, qoff = _eq_call(x, idx_flat)
    agg = _a_call(qlist, qoff, esum)
    return _matmul_call(agg.reshape(N_PAD, D), weight)


# speedup vs baseline: 1.1840x; 1.1840x over previous
import functools

import jax
import jax.numpy as jnp
from jax import lax
from jax.experimental import pallas as pl
from jax.experimental.pallas import tpu as pltpu
from jax.experimental.pallas import tpu_sc as plsc

N_NODES = 50000
D = 256
H_EDGES = 37500
K = 8

NC = 2
NS = 16
NW = NC * NS
L = 16

E_BLK = 16
EW = 1184
H_PAD = NW * EW
NBLK = EW // E_BLK
NBLK2 = NBLK // 2
IDX_PAD = H_PAD * K
CH = IDX_PAD // NW
QD = 4
QBUF = CH + 32
CAP = 16384
SENT = 0xFFFF

NR = 128
RNG = 392
N_PAD = NR * RNG
BR = RNG + 8
QSPAN = N_PAD // QD
PIECE = 2048
SBUF = PIECE + 32 + L
G = 32

BM = 400


def _eq_body(x_hbm, idx_hbm, esum_hbm, qlist_hbm, qoff_hbm,
             idxc_v, rows0_v, rows1_v, es_v, qbuf_v, qvec_v, sg0, sg1):
    cid = lax.axis_index("c")
    sid = lax.axis_index("s")
    wid = sid * NC + cid
    base = wid * CH
    lane = lax.iota(jnp.int32, L)

    pltpu.sync_copy(idx_hbm.at[pl.ds(base, CH)], idxc_v)

    def sinit(i, c):
        qbuf_v[pl.ds(i * L, L)] = jnp.full((L,), SENT, jnp.int32)
        return c
    lax.fori_loop(0, QBUF // L + 1, sinit, 0)
    pltpu.sync_copy(qbuf_v.at[pl.ds(0, 4096)],
                    qlist_hbm.at[pl.ds(wid * CAP + QBUF, 4096)])
    pltpu.sync_copy(qbuf_v.at[pl.ds(0, CAP - QBUF - 4096)],
                    qlist_hbm.at[pl.ds(wid * CAP + QBUF + 4096,
                                       CAP - QBUF - 4096)])

    offs = [jnp.int32(0)]
    ptr = jnp.int32(0)
    for q in range(QD):
        qlo = q * QSPAN
        qhi = (q + 1) * QSPAN

        def scan(i, p):
            v = idxc_v[pl.ds(i * L, L)]
            h = (base + i * L + lane) >> 3
            m = (v >= qlo) & (v < qhi) & (h < H_EDGES)
            word = (h << 16) | v
            plsc.store_compressed(qbuf_v.at[pl.ds(p, L)], word, mask=m)
            return p + plsc.all_reduce_population_count(m)[0]

        ptr = lax.fori_loop(0, CH // L, scan, ptr)
        ptr = (ptr + 7) & ~7
        offs.append(ptr)

    qvec = jnp.zeros((L,), jnp.int32)
    for i, o in enumerate(offs):
        qvec = jnp.where(lane == i, o, qvec)
    qvec_v[pl.ds(0, L)] = qvec
    pltpu.sync_copy(qvec_v, qoff_hbm.at[pl.ds(wid * L, L)])
    pltpu.sync_copy(qbuf_v.at[pl.ds(0, QBUF)],
                    qlist_hbm.at[pl.ds(wid * CAP, QBUF)])

    rows_v = (rows0_v, rows1_v)
    sg = (sg0, sg1)

    def fire_gather(b, p):
        pltpu.async_copy(
            x_hbm.at[idxc_v.at[pl.ds(b * E_BLK * K, E_BLK * K)]],
            rows_v[p], sg[p])

    def drain_gather(p):
        pltpu.make_async_copy(
            x_hbm.at[idxc_v.at[pl.ds(0, E_BLK * K)]],
            rows_v[p], sg[p]).wait()

    def process(b, p, b2):
        if p == 0:
            fire_gather(b + 1, 1)
        else:
            @pl.when(b2 <= NBLK2 - 2)
            def _():
                fire_gather(b + 1, 0)
        drain_gather(p)

        def edge(e, c2):
            r0 = e * K
            for j in range(D // L):
                acc = rows_v[p][r0, pl.ds(j * L, L)]
                for k in range(1, K):
                    acc = acc + rows_v[p][r0 + k, pl.ds(j * L, L)]
                es_v[e, pl.ds(j * L, L)] = acc
            return c2
        lax.fori_loop(0, E_BLK, edge, 0)
        pltpu.sync_copy(es_v, esum_hbm.at[pl.ds(wid * EW + b * E_BLK, E_BLK)])

    def pair(b2, carry):
        process(2 * b2, 0, b2)
        process(2 * b2 + 1, 1, b2)
        return carry

    fire_gather(0, 0)
    lax.fori_loop(0, NBLK2, pair, 0)


_eq_call = functools.partial(
    pl.kernel,
    out_type=(
        jax.ShapeDtypeStruct((H_PAD, D), jnp.float32),
        jax.ShapeDtypeStruct((NW * CAP,), jnp.int32),
        jax.ShapeDtypeStruct((NW * L,), jnp.int32),
    ),
    mesh=plsc.VectorSubcoreMesh(core_axis_name="c", subcore_axis_name="s"),
    scratch_types=[
        pltpu.VMEM((CH,), jnp.int32),
        pltpu.VMEM((E_BLK * K, D), jnp.float32),
        pltpu.VMEM((E_BLK * K, D), jnp.float32),
        pltpu.VMEM((E_BLK, D), jnp.float32),
        pltpu.VMEM((QBUF + L,), jnp.int32),
        pltpu.VMEM((L,), jnp.int32),
        pltpu.SemaphoreType.DMA,
        pltpu.SemaphoreType.DMA,
    ],
    compiler_params=pltpu.CompilerParams(needs_layout_passes=False),
)(_eq_body)


def _a_body(qlist_hbm, qoff_hbm, esum_hbm, agg_hbm,
            qot_v, qix_v, piece0_v, piece1_v, stage_v, rows0_v, rows1_v,
            gidx0_v, gidx1_v, bin_v, sp0, sp1, sa0, sa1):
    cid = lax.axis_index("c")
    sid = lax.axis_index("s")
    wid = sid * NC + cid
    lane = lax.iota(jnp.int32, L)

    piece_v = (piece0_v, piece1_v)
    rows_v = (rows0_v, rows1_v)
    gidx_v = (gidx0_v, gidx1_v)
    sp = (sp0, sp1)
    sa = (sa0, sa1)

    pltpu.sync_copy(qoff_hbm, qot_v)

    for rr0 in range(QD):
        def reidx(w2, c):
            ov = qot_v[pl.ds(w2 * L, L)]
            pair = jnp.where(lane == 0, ov[rr0],
                             jnp.where(lane == 1, ov[rr0 + 1], 0))
            qix_v[pl.ds((rr0 * NW + w2) * L, L)] = pair
            return c
        lax.fori_loop(0, NW, reidx, 0)

    def range_body(rr, carry0):
        lo = (wid + rr * NW) * RNG

        def zero(i, c):
            for j in range(D // L):
                bin_v[pl.ds(i * D + j * L, L)] = jnp.zeros((L,), jnp.float32)
            return c
        lax.fori_loop(0, RNG, zero, 0)

        def fire_piece(w2, pi, q):
            ov = qix_v[pl.ds((rr * NW + w2) * L, L)]
            start = pl.multiple_of(w2 * CAP + ov[0] + pi * PIECE, 8)
            pltpu.async_copy(qlist_hbm.at[pl.ds(start, PIECE)],
                             piece_v[q], sp[q])

        def drain_piece(q):
            pltpu.make_async_copy(qlist_hbm.at[pl.ds(0, PIECE)],
                                  piece_v[q], sp[q]).wait()

        def fire_rows(b, q):
            for t in range(G // L):
                w = stage_v[pl.ds(b * G + t * L, L)]
                gidx_v[q][pl.ds(t * L, L)] = (w >> 16) & 0xFFFF
            pltpu.async_copy(esum_hbm.at[gidx_v[q]], rows_v[q], sa[q])

        def drain_rows(q):
            pltpu.make_async_copy(esum_hbm.at[gidx_v[q]],
                                  rows_v[q], sa[q]).wait()

        def accum(b, q):
            for t in range(G // L):
                w = stage_v[pl.ds(b * G + t * L, L)]
                offv = (w & 0xFFFF) - lo

                def row(j2, c4):
                    s = offv.at[jnp.full((L,), j2, jnp.int32)].get(
                        mode="promise_in_bounds")[0]
                    for jj in range(D // L):
                        plsc.addupdate(
                            bin_v.at[pl.ds(s * D + jj * L, L)],
                            rows_v[q][t * L + j2, pl.ds(jj * L, L)])
                    return c4
                lax.fori_loop(0, L, row, 0)

        def process_piece(q):
            def scan(i, p):
                w = piece_v[q][pl.ds(i * L, L)]
                v = w & 0xFFFF
                m = (v >= lo) & (v < lo + RNG)
                plsc.store_compressed(stage_v.at[pl.ds(p, L)], w, mask=m)
                return p + plsc.all_reduce_population_count(m)[0]

            mtot = lax.fori_loop(0, PIECE // L, scan, 0)
            pad = ((lane & 7) << 16) | (lo + RNG + (lane & 7))
            for t in range(G // L):
                plsc.store_compressed(
                    stage_v.at[pl.ds(mtot + t * L, L)], pad, mask=lane < L)
            nb = (mtot + G - 1) // G

            @pl.when(nb >= 1)
            def _():
                fire_rows(0, 0)

            def blk(bi, c3):
                even = (bi & 1) == 0

                @pl.when((bi + 1 < nb) & even)
                def _():
                    fire_rows(bi + 1, 1)

                @pl.when((bi + 1 < nb) & (~even))
                def _():
                    fire_rows(bi + 1, 0)

                @pl.when(even)
                def _():
                    drain_rows(0)
                    accum(bi, 0)

                @pl.when(~even)
                def _():
                    drain_rows(1)
                    accum(bi, 1)
                return c3

            lax.fori_loop(0, nb, blk, 0)

        def src(w2, c):
            nq = (w2 + 1) & 1

            @pl.when(((w2 & 1) == 0) & (w2 + 1 < NW))
            def _():
                fire_piece(w2 + 1, 0, 1)

            @pl.when(((w2 & 1) == 1) & (w2 + 1 < NW))
            def _():
                fire_piece(w2 + 1, 0, 0)

            ov = qix_v[pl.ds((rr * NW + w2) * L, L)]
            npiece = (ov[1] - ov[0] + PIECE - 1) // PIECE

            @pl.when((w2 & 1) == 0)
            def _():
                drain_piece(0)

                @pl.when(npiece >= 1)
                def _():
                    process_piece(0)

                def extra(pi, c2):
                    fire_piece(w2, pi, 0)
                    drain_piece(0)
                    process_piece(0)
                    return c2
                lax.fori_loop(1, npiece, extra, 0)

            @pl.when((w2 & 1) == 1)
            def _():
                drain_piece(1)

                @pl.when(npiece >= 1)
                def _():
                    process_piece(1)

                def extra(pi, c2):
                    fire_piece(w2, pi, 1)
                    drain_piece(1)
                    process_piece(1)
                    return c2
                lax.fori_loop(1, npiece, extra, 0)
            return c

        fire_piece(0, 0, 0)
        lax.fori_loop(0, NW, src, 0)

        pltpu.sync_copy(bin_v.at[pl.ds(0, RNG * D)],
                        agg_hbm.at[pl.ds(lo * D, RNG * D)])
        return carry0

    lax.fori_loop(0, QD, range_body, 0)


_a_call = functools.partial(
    pl.kernel,
    out_type=jax.ShapeDtypeStruct((N_PAD * D,), jnp.float32),
    mesh=plsc.VectorSubcoreMesh(core_axis_name="c", subcore_axis_name="s"),
    scratch_types=[
        pltpu.VMEM((NW * L,), jnp.int32),
        pltpu.VMEM((QD * NW * L,), jnp.int32),
        pltpu.VMEM((PIECE,), jnp.int32),
        pltpu.VMEM((PIECE,), jnp.int32),
        pltpu.VMEM((SBUF,), jnp.int32),
        pltpu.VMEM((G, D), jnp.float32),
        pltpu.VMEM((G, D), jnp.float32),
        pltpu.VMEM((G,), jnp.int32),
        pltpu.VMEM((G,), jnp.int32),
        pltpu.VMEM((BR * D,), jnp.float32),
        pltpu.SemaphoreType.DMA,
        pltpu.SemaphoreType.DMA,
        pltpu.SemaphoreType.DMA,
        pltpu.SemaphoreType.DMA,
    ],
    compiler_params=pltpu.CompilerParams(needs_layout_passes=False),
)(_a_body)


def _matmul_body(a_ref, w_ref, o_ref):
    o_ref[...] = jnp.maximum(
        jnp.dot(a_ref[...], w_ref[...], preferred_element_type=jnp.float32),
        0.0)


_matmul_call = pl.pallas_call(
    _matmul_body,
    grid=(N_NODES // BM,),
    in_specs=[
        pl.BlockSpec((BM, D), lambda i: (i, 0)),
        pl.BlockSpec((D, D), lambda i: (0, 0)),
    ],
    out_specs=pl.BlockSpec((BM, D), lambda i: (i, 0)),
    out_shape=jax.ShapeDtypeStruct((N_NODES, D), jnp.float32),
)


def kernel(x, hyperedges, weight):
    idx_flat = jnp.pad(hyperedges.reshape(-1), (0, IDX_PAD - H_EDGES * K))
    esum, qlist, qoff = _eq_call(x, idx_flat)
    agg = _a_call(qlist, qoff, esum)
    return _matmul_call(agg.reshape(N_PAD, D), weight)


# speedup vs baseline: 1.2710x; 1.0735x over previous
import functools

import jax
import jax.numpy as jnp
from jax import lax
from jax.experimental import pallas as pl
from jax.experimental.pallas import tpu as pltpu
from jax.experimental.pallas import tpu_sc as plsc

N_NODES = 50000
D = 256
H_EDGES = 37500
K = 8

NC = 2
NS = 16
NW = NC * NS
L = 16

E_BLK = 16
EW = 1184
H_PAD = NW * EW
NBLK = EW // E_BLK
NBLK2 = NBLK // 2
IDX_PAD = H_PAD * K
CH = IDX_PAD // NW
QD = 4
QBUF = CH + 32
CAP = 16384
SENT = 0xFFFF

NR = 128
RNG = 392
N_PAD = NR * RNG
BR = RNG + 8
QSPAN = N_PAD // QD
PIECE = 2048
SBUF = PIECE + 32 + L
G = 32

BM = 400


def _eq_body(x_hbm, idx_hbm, esum_hbm, qlist_hbm, qoff_hbm,
             idxc_v, rows0_v, rows1_v, es_v, qbuf_v, qvec_v, sg0, sg1):
    cid = lax.axis_index("c")
    sid = lax.axis_index("s")
    wid = sid * NC + cid
    base = wid * CH
    lane = lax.iota(jnp.int32, L)

    pltpu.sync_copy(idx_hbm.at[pl.ds(base, CH)], idxc_v)

    def sinit(i, c):
        qbuf_v[pl.ds(i * L, L)] = jnp.full((L,), SENT, jnp.int32)
        return c
    lax.fori_loop(0, QBUF // L + 1, sinit, 0)
    pltpu.sync_copy(qbuf_v.at[pl.ds(0, 4096)],
                    qlist_hbm.at[pl.ds(wid * CAP + QBUF, 4096)])
    pltpu.sync_copy(qbuf_v.at[pl.ds(0, CAP - QBUF - 4096)],
                    qlist_hbm.at[pl.ds(wid * CAP + QBUF + 4096,
                                       CAP - QBUF - 4096)])

    offs = [jnp.int32(0)]
    ptr = jnp.int32(0)
    for q in range(QD):
        qlo = q * QSPAN
        qhi = (q + 1) * QSPAN

        def scan(i, p):
            v = idxc_v[pl.ds(i * L, L)]
            h = (base + i * L + lane) >> 3
            m = (v >= qlo) & (v < qhi) & (h < H_EDGES)
            word = (h << 16) | v
            plsc.store_compressed(qbuf_v.at[pl.ds(p, L)], word, mask=m)
            return p + plsc.all_reduce_population_count(m)[0]

        ptr = lax.fori_loop(0, CH // L, scan, ptr)
        ptr = (ptr + 7) & ~7
        offs.append(ptr)

    qvec = jnp.zeros((L,), jnp.int32)
    for i, o in enumerate(offs):
        qvec = jnp.where(lane == i, o, qvec)
    qvec_v[pl.ds(0, L)] = qvec
    pltpu.sync_copy(qvec_v, qoff_hbm.at[pl.ds(wid * L, L)])
    pltpu.sync_copy(qbuf_v.at[pl.ds(0, QBUF)],
                    qlist_hbm.at[pl.ds(wid * CAP, QBUF)])

    rows_v = (rows0_v, rows1_v)
    sg = (sg0, sg1)

    def fire_gather(b, p):
        pltpu.async_copy(
            x_hbm.at[idxc_v.at[pl.ds(b * E_BLK * K, E_BLK * K)]],
            rows_v[p], sg[p])

    def drain_gather(p):
        pltpu.make_async_copy(
            x_hbm.at[idxc_v.at[pl.ds(0, E_BLK * K)]],
            rows_v[p], sg[p]).wait()

    def process(b, p, b2):
        if p == 0:
            fire_gather(b + 1, 1)
        else:
            @pl.when(b2 <= NBLK2 - 2)
            def _():
                fire_gather(b + 1, 0)
        drain_gather(p)

        def edge(e, c2):
            r0 = e * K
            for j in range(D // L):
                acc = rows_v[p][r0, pl.ds(j * L, L)]
                for k in range(1, K):
                    acc = acc + rows_v[p][r0 + k, pl.ds(j * L, L)]
                es_v[e, pl.ds(j * L, L)] = acc
            return c2
        lax.fori_loop(0, E_BLK, edge, 0)
        pltpu.sync_copy(es_v, esum_hbm.at[pl.ds(wid * EW + b * E_BLK, E_BLK)])

    def pair(b2, carry):
        process(2 * b2, 0, b2)
        process(2 * b2 + 1, 1, b2)
        return carry

    fire_gather(0, 0)
    lax.fori_loop(0, NBLK2, pair, 0)


_eq_call = functools.partial(
    pl.kernel,
    out_type=(
        jax.ShapeDtypeStruct((H_PAD, D), jnp.float32),
        jax.ShapeDtypeStruct((NW * CAP,), jnp.int32),
        jax.ShapeDtypeStruct((NW * L,), jnp.int32),
    ),
    mesh=plsc.VectorSubcoreMesh(core_axis_name="c", subcore_axis_name="s"),
    scratch_types=[
        pltpu.VMEM((CH,), jnp.int32),
        pltpu.VMEM((E_BLK * K, D), jnp.float32),
        pltpu.VMEM((E_BLK * K, D), jnp.float32),
        pltpu.VMEM((E_BLK, D), jnp.float32),
        pltpu.VMEM((QBUF + L,), jnp.int32),
        pltpu.VMEM((L,), jnp.int32),
        pltpu.SemaphoreType.DMA,
        pltpu.SemaphoreType.DMA,
    ],
    compiler_params=pltpu.CompilerParams(needs_layout_passes=False),
)(_eq_body)


def _a_body(qlist_hbm, qoff_hbm, esum_hbm, agg_hbm,
            qot_v, qix_v, piece0_v, piece1_v, stage_v, rows0_v, rows1_v,
            gidx0_v, gidx1_v, bin_v, sp0, sp1, sa0, sa1):
    cid = lax.axis_index("c")
    sid = lax.axis_index("s")
    wid = sid * NC + cid
    lane = lax.iota(jnp.int32, L)

    piece_v = (piece0_v, piece1_v)
    rows_v = (rows0_v, rows1_v)
    gidx_v = (gidx0_v, gidx1_v)
    sp = (sp0, sp1)
    sa = (sa0, sa1)

    pltpu.sync_copy(qoff_hbm, qot_v)

    for rr0 in range(QD):
        def reidx(w2, c):
            ov = qot_v[pl.ds(w2 * L, L)]
            pair = jnp.where(lane == 0, ov[rr0],
                             jnp.where(lane == 1, ov[rr0 + 1], 0))
            qix_v[pl.ds((rr0 * NW + w2) * L, L)] = pair
            return c
        lax.fori_loop(0, NW, reidx, 0)

    def range_body(rr, carry0):
        lo = (wid + rr * NW) * RNG

        def zero(i, c):
            for j in range(D // L):
                bin_v[pl.ds(i * D + j * L, L)] = jnp.zeros((L,), jnp.float32)
            return c
        lax.fori_loop(0, RNG, zero, 0)

        def fire_piece(w2, pi, q):
            ov = qix_v[pl.ds((rr * NW + w2) * L, L)]
            start = pl.multiple_of(w2 * CAP + ov[0] + pi * PIECE, 8)
            pltpu.async_copy(qlist_hbm.at[pl.ds(start, PIECE)],
                             piece_v[q], sp[q])

        def drain_piece(q):
            pltpu.make_async_copy(qlist_hbm.at[pl.ds(0, PIECE)],
                                  piece_v[q], sp[q]).wait()

        def fire_rows(b, q):
            for t in range(G // L):
                w = stage_v[pl.ds(b * G + t * L, L)]
                gidx_v[q][pl.ds(t * L, L)] = (w >> 16) & 0xFFFF
            pltpu.async_copy(esum_hbm.at[gidx_v[q]], rows_v[q], sa[q])

        def drain_rows(q):
            pltpu.make_async_copy(esum_hbm.at[gidx_v[q]],
                                  rows_v[q], sa[q]).wait()

        def accum(b, q):
            for t in range(G // L):
                w = stage_v[pl.ds(b * G + t * L, L)]
                offv = (w & 0xFFFF) - lo

                def row(j2, c4):
                    s = offv.at[jnp.full((L,), j2, jnp.int32)].get(
                        mode="promise_in_bounds")[0]
                    for jj in range(D // L):
                        plsc.addupdate(
                            bin_v.at[pl.ds(s * D + jj * L, L)],
                            rows_v[q][t * L + j2, pl.ds(jj * L, L)])
                    return c4
                lax.fori_loop(0, L, row, 0)

        def process_piece(q, nwords):
            def scan(i, p):
                w = piece_v[q][pl.ds(i * L, L)]
                v = w & 0xFFFF
                m = (v >= lo) & (v < lo + RNG)
                plsc.store_compressed(stage_v.at[pl.ds(p, L)], w, mask=m)
                return p + plsc.all_reduce_population_count(m)[0]

            mtot = lax.fori_loop(0, (nwords + L - 1) // L, scan, 0)
            pad = ((lane & 7) << 16) | (lo + RNG + (lane & 7))
            for t in range(G // L):
                plsc.store_compressed(
                    stage_v.at[pl.ds(mtot + t * L, L)], pad, mask=lane < L)
            nb = (mtot + G - 1) // G

            @pl.when(nb >= 1)
            def _():
                fire_rows(0, 0)

            def blk(bi, c3):
                even = (bi & 1) == 0

                @pl.when((bi + 1 < nb) & even)
                def _():
                    fire_rows(bi + 1, 1)

                @pl.when((bi + 1 < nb) & (~even))
                def _():
                    fire_rows(bi + 1, 0)

                @pl.when(even)
                def _():
                    drain_rows(0)
                    accum(bi, 0)

                @pl.when(~even)
                def _():
                    drain_rows(1)
                    accum(bi, 1)
                return c3

            lax.fori_loop(0, nb, blk, 0)

        def src(w2, c):
            nq = (w2 + 1) & 1

            @pl.when(((w2 & 1) == 0) & (w2 + 1 < NW))
            def _():
                fire_piece(w2 + 1, 0, 1)

            @pl.when(((w2 & 1) == 1) & (w2 + 1 < NW))
            def _():
                fire_piece(w2 + 1, 0, 0)

            ov = qix_v[pl.ds((rr * NW + w2) * L, L)]
            len8 = ov[1] - ov[0]
            npiece = (len8 + PIECE - 1) // PIECE

            @pl.when((w2 & 1) == 0)
            def _():
                drain_piece(0)

                @pl.when(npiece >= 1)
                def _():
                    process_piece(0, jnp.minimum(len8, PIECE))

                def extra(pi, c2):
                    fire_piece(w2, pi, 0)
                    drain_piece(0)
                    process_piece(0, jnp.minimum(len8 - pi * PIECE, PIECE))
                    return c2
                lax.fori_loop(1, npiece, extra, 0)

            @pl.when((w2 & 1) == 1)
            def _():
                drain_piece(1)

                @pl.when(npiece >= 1)
                def _():
                    process_piece(1, jnp.minimum(len8, PIECE))

                def extra(pi, c2):
                    fire_piece(w2, pi, 1)
                    drain_piece(1)
                    process_piece(1, jnp.minimum(len8 - pi * PIECE, PIECE))
                    return c2
                lax.fori_loop(1, npiece, extra, 0)
            return c

        fire_piece(0, 0, 0)
        lax.fori_loop(0, NW, src, 0)

        pltpu.sync_copy(bin_v.at[pl.ds(0, RNG * D)],
                        agg_hbm.at[pl.ds(lo * D, RNG * D)])
        return carry0

    lax.fori_loop(0, QD, range_body, 0)


_a_call = functools.partial(
    pl.kernel,
    out_type=jax.ShapeDtypeStruct((N_PAD * D,), jnp.float32),
    mesh=plsc.VectorSubcoreMesh(core_axis_name="c", subcore_axis_name="s"),
    scratch_types=[
        pltpu.VMEM((NW * L,), jnp.int32),
        pltpu.VMEM((QD * NW * L,), jnp.int32),
        pltpu.VMEM((PIECE,), jnp.int32),
        pltpu.VMEM((PIECE,), jnp.int32),
        pltpu.VMEM((SBUF,), jnp.int32),
        pltpu.VMEM((G, D), jnp.float32),
        pltpu.VMEM((G, D), jnp.float32),
        pltpu.VMEM((G,), jnp.int32),
        pltpu.VMEM((G,), jnp.int32),
        pltpu.VMEM((BR * D,), jnp.float32),
        pltpu.SemaphoreType.DMA,
        pltpu.SemaphoreType.DMA,
        pltpu.SemaphoreType.DMA,
        pltpu.SemaphoreType.DMA,
    ],
    compiler_params=pltpu.CompilerParams(needs_layout_passes=False),
)(_a_body)


def _matmul_body(a_ref, w_ref, o_ref):
    o_ref[...] = jnp.maximum(
        jnp.dot(a_ref[...], w_ref[...], preferred_element_type=jnp.float32),
        0.0)


_matmul_call = pl.pallas_call(
    _matmul_body,
    grid=(N_NODES // BM,),
    in_specs=[
        pl.BlockSpec((BM, D), lambda i: (i, 0)),
        pl.BlockSpec((D, D), lambda i: (0, 0)),
    ],
    out_specs=pl.BlockSpec((BM, D), lambda i: (i, 0)),
    out_shape=jax.ShapeDtypeStruct((N_NODES, D), jnp.float32),
)


def kernel(x, hyperedges, weight):
    idx_flat = jnp.pad(hyperedges.reshape(-1), (0, IDX_PAD - H_EDGES * K))
    esum, qlist, qoff = _eq_call(x, idx_flat)
    agg = _a_call(qlist, qoff, esum)
    return _matmul_call(agg.reshape(N_PAD, D), weight)
